# baseline (device time: 243563 ns/iter reference)
import functools

import jax
import jax.numpy as jnp
from jax import lax
from jax.experimental import pallas as pl
from jax.experimental.pallas import tpu as pltpu

N_CHUNK = 16
SLOTS = 3
LAG = 2


def kernel(x, dy):
    m, d = x.shape
    _, f = dy.shape
    dh = d // 2
    fh = f // 2
    ch = fh // N_CHUNK
    dn = (((0,), (0,)), ((), ()))

    def body(x_ref, dy_ref, out_ref, dy_buf, q_buf, dy_sems, s1, r1, s2, r2):
        mx = lax.axis_index("x")
        my = lax.axis_index("y")
        xn = (1 - mx, my)
        yn = (mx, 1 - my)
        col0 = my * fh
        keep0 = mx * dh
        send0 = (1 - mx) * dh

        def dy_dma(c):
            return pltpu.make_async_copy(
                dy_ref.at[:, pl.ds(col0 + c * ch, ch)],
                dy_buf.at[c % SLOTS],
                dy_sems.at[c % SLOTS],
            )

        dy_dmas = []
        for c in range(SLOTS - 1):
            dma = dy_dma(c)
            dma.start()
            dy_dmas.append(dma)

        barrier_sem = pltpu.get_barrier_semaphore()
        for nbr in (xn, yn):
            pl.semaphore_signal(
                barrier_sem, inc=1, device_id=nbr,
                device_id_type=pl.DeviceIdType.MESH,
            )
        pl.semaphore_wait(barrier_sem, 2)

        rdma1s = []
        rdma2s = []

        def handle_arrival(j):
            rdma1s[j].wait_recv()
            out_ref[:, pl.ds(col0 + j * ch, ch)] = (
                out_ref[:, pl.ds(col0 + j * ch, ch)]
                + q_buf[j % SLOTS, pl.ds(keep0, dh), :]
            )
            rdma2 = pltpu.make_async_remote_copy(
                src_ref=out_ref.at[:, pl.ds(col0 + j * ch, ch)],
                dst_ref=out_ref.at[:, pl.ds(col0 + j * ch, ch)],
                send_sem=s2.at[j],
                recv_sem=r2.at[j],
                device_id=yn,
                device_id_type=pl.DeviceIdType.MESH,
            )
            rdma2.start()
            rdma2s.append(rdma2)

        for c in range(N_CHUNK):
            if c + SLOTS - 1 < N_CHUNK:
                dma = dy_dma(c + SLOTS - 1)
                dma.start()
                dy_dmas.append(dma)
            if c >= SLOTS:
                rdma1s[c - SLOTS].wait_send()
            if c >= LAG:
                handle_arrival(c - LAG)
            dy_dmas[c].wait()
            q_buf[c % SLOTS] = lax.dot_general(
                x_ref[:, :], dy_buf[c % SLOTS], dn,
                preferred_element_type=jnp.float32,
            )
            rdma1 = pltpu.make_async_remote_copy(
                src_ref=q_buf.at[c % SLOTS, pl.ds(send0, dh), :],
                dst_ref=out_ref.at[:, pl.ds(col0 + c * ch, ch)],
                send_sem=s1.at[c],
                recv_sem=r1.at[c],
                device_id=xn,
                device_id_type=pl.DeviceIdType.MESH,
            )
            rdma1.start()
            rdma1s.append(rdma1)

        for j in range(N_CHUNK - LAG, N_CHUNK):
            handle_arrival(j)

        for j in range(N_CHUNK):
            rdma2s[j].wait_recv()
        for j in range(N_CHUNK - SLOTS, N_CHUNK):
            rdma1s[j].wait_send()
        for j in range(N_CHUNK):
            rdma2s[j].wait_send()

        @functools.partial(
            pl.run_scoped, sem2=pltpu.SemaphoreType.REGULAR
        )
        def _(sem2):
            for nbr in (xn, yn):
                pl.semaphore_signal(
                    sem2, inc=1, device_id=nbr,
                    device_id_type=pl.DeviceIdType.MESH,
                )
            pl.semaphore_wait(sem2, 2)

    return pl.pallas_call(
        body,
        out_shape=jax.ShapeDtypeStruct((dh, f), jnp.float32),
        in_specs=[
            pl.BlockSpec(memory_space=pltpu.VMEM),
            pl.BlockSpec(memory_space=pl.ANY),
        ],
        out_specs=pl.BlockSpec(memory_space=pltpu.VMEM),
        scratch_shapes=[
            pltpu.VMEM((SLOTS, m, ch), jnp.float32),
            pltpu.VMEM((SLOTS, d, ch), jnp.float32),
            pltpu.SemaphoreType.DMA((SLOTS,)),
            pltpu.SemaphoreType.DMA((N_CHUNK,)),
            pltpu.SemaphoreType.DMA((N_CHUNK,)),
            pltpu.SemaphoreType.DMA((N_CHUNK,)),
            pltpu.SemaphoreType.DMA((N_CHUNK,)),
        ],
        compiler_params=pltpu.CompilerParams(
            collective_id=0, vmem_limit_bytes=64 * 1024 * 1024
        ),
    )(x, dy)


# device time: 233108 ns/iter; 1.0449x vs baseline; 1.0449x over previous
import contextlib
import functools
import os

import jax
import jax.numpy as jnp
from jax import lax
from jax.experimental import pallas as pl
from jax.experimental.pallas import tpu as pltpu

N_CHUNK = 16
SLOTS = 3
LAG = 2

_PROFILE = os.environ.get("KERNEL_PROFILE", "0") == "1"


def _scope(name):
    return jax.named_scope(name) if _PROFILE else contextlib.nullcontext()


def kernel(x, dy):
    m, d = x.shape
    _, f = dy.shape
    dh = d // 2
    fh = f // 2
    ch = fh // N_CHUNK
    dn = (((0,), (0,)), ((), ()))

    def body(x_ref, dy_ref, out_ref, acc, dy_buf, q_buf,
             dy_sems, osems, s1, r1, s2, r2):
        mx = lax.axis_index("x")
        my = lax.axis_index("y")
        xn = (1 - mx, my)
        yn = (mx, 1 - my)
        col0 = my * fh
        keep0 = mx * dh
        send0 = (1 - mx) * dh

        def dy_dma(c):
            return pltpu.make_async_copy(
                dy_ref.at[:, pl.ds(col0 + c * ch, ch)],
                dy_buf.at[c % SLOTS],
                dy_sems.at[c % SLOTS],
            )

        dy_dmas = []
        for c in range(SLOTS - 1):
            dma = dy_dma(c)
            dma.start()
            dy_dmas.append(dma)

        barrier_sem = pltpu.get_barrier_semaphore()
        for nbr in (xn, yn):
            pl.semaphore_signal(
                barrier_sem, inc=1, device_id=nbr,
                device_id_type=pl.DeviceIdType.MESH,
            )
        pl.semaphore_wait(barrier_sem, 2)

        rdma1s = []
        rdma2s = []
        odmas = []

        def handle_arrival(j):
            with _scope(f"wr1#c={j}"):
                rdma1s[j].wait_recv()
            with _scope(f"add#c={j}"):
                acc[:, pl.ds(j * ch, ch)] = (
                    acc[:, pl.ds(j * ch, ch)]
                    + q_buf[j % SLOTS, pl.ds(keep0, dh), :]
                )
            rdma2 = pltpu.make_async_remote_copy(
                src_ref=acc.at[:, pl.ds(j * ch, ch)],
                dst_ref=out_ref.at[:, pl.ds(col0 + j * ch, ch)],
                send_sem=s2.at[j],
                recv_sem=r2.at[j],
                device_id=yn,
                device_id_type=pl.DeviceIdType.MESH,
            )
            rdma2.start()
            rdma2s.append(rdma2)
            odma = pltpu.make_async_copy(
                acc.at[:, pl.ds(j * ch, ch)],
                out_ref.at[:, pl.ds(col0 + j * ch, ch)],
                osems.at[j],
            )
            odma.start()
            odmas.append(odma)

        for c in range(N_CHUNK):
            if c + SLOTS - 1 < N_CHUNK:
                dma = dy_dma(c + SLOTS - 1)
                dma.start()
                dy_dmas.append(dma)
            if c >= SLOTS:
                with _scope(f"ws1#c={c}"):
                    rdma1s[c - SLOTS].wait_send()
            if c >= LAG:
                handle_arrival(c - LAG)
            with _scope(f"dyw#c={c}"):
                dy_dmas[c].wait()
            with _scope(f"gemm#c={c}"):
                q_buf[c % SLOTS] = lax.dot_general(
                    x_ref[:, :], dy_buf[c % SLOTS], dn,
                    preferred_element_type=jnp.float32,
                )
            rdma1 = pltpu.make_async_remote_copy(
                src_ref=q_buf.at[c % SLOTS, pl.ds(send0, dh), :],
                dst_ref=acc.at[:, pl.ds(c * ch, ch)],
                send_sem=s1.at[c],
                recv_sem=r1.at[c],
                device_id=xn,
                device_id_type=pl.DeviceIdType.MESH,
            )
            rdma1.start()
            rdma1s.append(rdma1)

        for j in range(N_CHUNK - LAG, N_CHUNK):
            handle_arrival(j)

        with _scope("wr2_drain"):
            for j in range(N_CHUNK):
                rdma2s[j].wait_recv()
                odmas[j].wait()
        with _scope("wsend_drain"):
            for j in range(N_CHUNK - SLOTS, N_CHUNK):
                rdma1s[j].wait_send()
            for j in range(N_CHUNK):
                rdma2s[j].wait_send()

        @functools.partial(
            pl.run_scoped, sem2=pltpu.SemaphoreType.REGULAR
        )
        def _(sem2):
            for nbr in (xn, yn):
                pl.semaphore_signal(
                    sem2, inc=1, device_id=nbr,
                    device_id_type=pl.DeviceIdType.MESH,
                )
            pl.semaphore_wait(sem2, 2)

    return pl.pallas_call(
        body,
        out_shape=jax.ShapeDtypeStruct((dh, f), jnp.float32),
        in_specs=[
            pl.BlockSpec(memory_space=pltpu.VMEM),
            pl.BlockSpec(memory_space=pl.ANY),
        ],
        out_specs=pl.BlockSpec(memory_space=pl.ANY),
        scratch_shapes=[
            pltpu.VMEM((dh, fh), jnp.float32),
            pltpu.VMEM((SLOTS, m, ch), jnp.float32),
            pltpu.VMEM((SLOTS, d, ch), jnp.float32),
            pltpu.SemaphoreType.DMA((SLOTS,)),
            pltpu.SemaphoreType.DMA((N_CHUNK,)),
            pltpu.SemaphoreType.DMA((N_CHUNK,)),
            pltpu.SemaphoreType.DMA((N_CHUNK,)),
            pltpu.SemaphoreType.DMA((N_CHUNK,)),
            pltpu.SemaphoreType.DMA((N_CHUNK,)),
        ],
        compiler_params=pltpu.CompilerParams(
            collective_id=0, vmem_limit_bytes=64 * 1024 * 1024
        ),
    )(x, dy)
